# baseline (speedup 1.0000x reference)
import functools

import jax
import jax.numpy as jnp
from jax import lax
from jax.experimental import pallas as pl
from jax.experimental.pallas import tpu as pltpu
from jax.experimental.pallas import tpu_sc as plsc

K_TOT = 32
DIL = 2
B = 4
N = 4096
D = 64
QB = 512
CH = 64
NCH = N // CH

NWORK = 32
RPW = (B * N) // NWORK
G = 8
NEG_INF = float("-inf")


def _dist_body(xq_ref, xk_ref, out_ref, mx_ref):
    q = xq_ref[0]
    k = xk_ref[0]
    inner = jax.lax.dot_general(
        q, k, (((1,), (1,)), ((), ())),
        preferred_element_type=jnp.float32,
    )
    x_inner = -2.0 * inner
    qsq = jnp.sum(q * q, axis=-1, keepdims=True)
    ksq = jnp.sum(k * k, axis=-1, keepdims=True)
    neg = -(qsq + x_inner + ksq.T)
    out_ref[...] = jnp.reshape(neg, (QB * (N // 128), 128))
    mx = jnp.max(jnp.reshape(neg, (QB, NCH, CH)), axis=2)
    mx_ref[...] = jnp.reshape(mx, (QB * NCH // 128, 128))


def _neg_adj(xb):
    return pl.pallas_call(
        _dist_body,
        grid=(B, N // QB),
        in_specs=[
            pl.BlockSpec((1, QB, D), lambda b, i: (b, i, 0)),
            pl.BlockSpec((1, N, D), lambda b, i: (b, 0, 0)),
        ],
        out_specs=[
            pl.BlockSpec(
                (QB * (N // 128), 128), lambda b, i: (b * (N // QB) + i, 0)),
            pl.BlockSpec(
                (QB * NCH // 128, 128), lambda b, i: (b * (N // QB) + i, 0)),
        ],
        out_shape=[
            jax.ShapeDtypeStruct((B * N * (N // 128), 128), jnp.float32),
            jax.ShapeDtypeStruct((B * N * NCH // 128, 128), jnp.float32),
        ],
    )(xb, xb)


_MESH = plsc.VectorSubcoreMesh(core_axis_name="c", subcore_axis_name="s")
_VPR = N // 16


def _merge32(av, ai, bv, bi):
    rbv1, rbi1 = lax.rev(bv[1], (0,)), lax.rev(bi[1], (0,))
    rbv0, rbi0 = lax.rev(bv[0], (0,)), lax.rev(bi[0], (0,))
    k0 = av[0] >= rbv1
    m0v = jnp.where(k0, av[0], rbv1)
    m0i = jnp.where(k0, ai[0], rbi1)
    k1 = av[1] >= rbv0
    m1v = jnp.where(k1, av[1], rbv0)
    m1i = jnp.where(k1, ai[1], rbi0)
    c = m0v >= m1v
    hv = jnp.where(c, m0v, m1v)
    hx = jnp.where(c, m0i, m1i)
    lv = jnp.where(c, m1v, m0v)
    lx = jnp.where(c, m1i, m0i)
    s0v, s0i = plsc.sort_key_val(hv, hx, descending=True)
    s1v, s1i = plsc.sort_key_val(lv, lx, descending=True)
    return (s0v, s1v), (s0i, s1i)


@functools.partial(
    pl.kernel,
    out_type=(
        jax.ShapeDtypeStruct((B * N * K_TOT,), jnp.float32),
        jax.ShapeDtypeStruct((B * N * K_TOT,), jnp.int32),
    ),
    mesh=_MESH,
    compiler_params=pltpu.CompilerParams(needs_layout_passes=False),
    scratch_types=[
        pltpu.VMEM((G * N,), jnp.float32),
        pltpu.VMEM((G * NCH,), jnp.float32),
        pltpu.VMEM((N + 16,), jnp.int32),
        pltpu.VMEM((RPW * K_TOT,), jnp.float32),
        pltpu.VMEM((RPW * K_TOT,), jnp.int32),
    ],
)
def _topk_sc(neg_hbm, mx_hbm, val_out, idx_out,
             inbuf, mxbuf, cand_i, outv, outi):
    cid = lax.axis_index("c")
    sid = lax.axis_index("s")
    wid = sid * 2 + cid
    row0 = wid * RPW
    lane = lax.iota(jnp.int32, 16)
    ninf16 = jnp.full((16,), NEG_INF, jnp.float32)
    zero16 = jnp.zeros((16,), jnp.int32)

    def do_row(rr, slot):
        rbase = rr * N

        mbase = rr * NCH
        s0, _ = plsc.sort_key_val(
            mxbuf[pl.ds(mbase, 16)], zero16, descending=True)
        s1, _ = plsc.sort_key_val(
            mxbuf[pl.ds(mbase + 16, 16)], zero16, descending=True)
        s2, _ = plsc.sort_key_val(
            mxbuf[pl.ds(mbase + 32, 16)], zero16, descending=True)
        s3, _ = plsc.sort_key_val(
            mxbuf[pl.ds(mbase + 48, 16)], zero16, descending=True)
        a, _ = _merge32((s0, ninf16), (zero16, zero16),
                        (s1, ninf16), (zero16, zero16))
        b, _ = _merge32((s2, ninf16), (zero16, zero16),
                        (s3, ninf16), (zero16, zero16))
        w, _ = _merge32(a, (zero16, zero16), b, (zero16, zero16))
        t = jnp.min(w[1])

        UB = 4

        def scan_b(i, off):
            for u in range(UB):
                j = i * UB + u
                v = inbuf[pl.ds(pl.multiple_of(rbase + j * 16, 16), 16)]
                mask = v >= t
                idxv = lane + j * 16
                plsc.store_compressed(cand_i.at[pl.ds(off, 16)], idxv, mask=mask)
                off = off + plsc.all_reduce_population_count(mask)[0]
            return off

        cnt = lax.fori_loop(0, _VPR // UB, scan_b, jnp.int32(0))
        cand_i[pl.ds(cnt, 16)] = zero16

        def merge(j, carry):
            b0v, b0i, b1v, b1i = carry
            ci = cand_i[pl.ds(pl.multiple_of(j * 16, 16), 16)]
            cv = plsc.load_gather(inbuf, [ci + rbase])
            valid = (lane + j * 16) < cnt
            cv = jnp.where(valid, cv, NEG_INF)
            cv, ci = plsc.sort_key_val(cv, ci, descending=True)
            (b0v, b1v), (b0i, b1i) = _merge32(
                (b0v, b1v), (b0i, b1i),
                (cv, ninf16), (ci, zero16))
            return b0v, b0i, b1v, b1i

        nch = (cnt + 15) // 16
        b0v, b0i, b1v, b1i = lax.fori_loop(
            0, nch, merge, (ninf16, zero16, ninf16, zero16))

        obase = slot * K_TOT
        outv[pl.ds(obase, 16)] = b0v
        outv[pl.ds(obase + 16, 16)] = b1v
        outi[pl.ds(obase, 16)] = b0i
        outi[pl.ds(obase + 16, 16)] = b1i

    def group(g, _):
        base = row0 + g * G
        pltpu.sync_copy(neg_hbm.at[pl.ds(base * N, G * N)], inbuf)
        pltpu.sync_copy(mx_hbm.at[pl.ds(base * NCH, G * NCH)], mxbuf)

        def row_body(rr, __):
            do_row(rr, g * G + rr)
            return __

        lax.fori_loop(0, G, row_body, 0)
        return _

    lax.fori_loop(0, RPW // G, group, 0)
    pltpu.sync_copy(outv, val_out.at[pl.ds(row0 * K_TOT, RPW * K_TOT)])
    pltpu.sync_copy(outi, idx_out.at[pl.ds(row0 * K_TOT, RPW * K_TOT)])


def kernel(x, batch):
    del batch
    xb = x.reshape(B, N, D)
    neg_flat, mx_flat = _neg_adj(xb)
    val, nn_idx = _topk_sc(neg_flat.reshape(B * N * N),
                           mx_flat.reshape(B * N * NCH))
    val = val.reshape(1, -1)
    start = (jnp.arange(B, dtype=jnp.int32) * N).reshape(B, 1, 1)
    nn_idx = (nn_idx.reshape(B, N, K_TOT) + start).reshape(1, -1)
    center = jnp.repeat(jnp.arange(B * N, dtype=jnp.int32), K_TOT).reshape(1, -1)
    edge_index = jnp.concatenate([nn_idx, center], axis=0)[:, ::DIL]
    return edge_index, val
